# baseline (speedup 1.0000x reference)
import functools

import jax
import jax.numpy as jnp
from jax import lax
from jax.experimental import pallas as pl
from jax.experimental.pallas import tpu as pltpu
from jax.experimental.pallas import tpu_sc as plsc

D_MODEL = 64
SCALE = 8.0

NC = 2
NS = 16
NW = NC * NS
CHUNK = 128
NBUF = 4


@functools.lru_cache(maxsize=None)
def _emb_call(B: int):
    b_per_w = B // NW
    n_chunks = b_per_w // CHUNK
    n_rounds = n_chunks // NBUF
    mesh = plsc.VectorSubcoreMesh(core_axis_name="c", subcore_axis_name="s")

    scratch = (
        [pltpu.VMEM((b_per_w,), jnp.int32)]
        + [pltpu.VMEM((CHUNK, D_MODEL), jnp.float32) for _ in range(2 * NBUF)]
        + [pltpu.SemaphoreType.DMA for _ in range(2 * NBUF)]
    )

    @functools.partial(
        pl.kernel,
        mesh=mesh,
        out_type=jax.ShapeDtypeStruct((B, D_MODEL), jnp.float32),
        scratch_types=scratch,
        compiler_params=pltpu.CompilerParams(use_tc_tiling_on_sc=False),
    )
    def emb(x_hbm, table_hbm, out_hbm, idx_v, *rest):
        gbuf = rest[:NBUF]
        sbuf = rest[NBUF:2 * NBUF]
        gsem = rest[2 * NBUF:3 * NBUF]
        ssem = rest[3 * NBUF:4 * NBUF]

        wid = lax.axis_index("s") * NC + lax.axis_index("c")
        base = wid * b_per_w
        pltpu.sync_copy(x_hbm.at[pl.ds(base, b_per_w)], idx_v)

        def start_gather(j, b):
            idx_slice = idx_v.at[pl.ds(j * CHUNK, CHUNK)]
            pltpu.async_copy(table_hbm.at[idx_slice], gbuf[b], gsem[b])

        def wait_gather(j, b):
            idx_slice = idx_v.at[pl.ds(j * CHUNK, CHUNK)]
            pltpu.make_async_copy(table_hbm.at[idx_slice], gbuf[b], gsem[b]).wait()

        def start_store(j, b):
            dst = out_hbm.at[pl.ds(base + j * CHUNK, CHUNK)]
            pltpu.async_copy(sbuf[b], dst, ssem[b])

        def wait_store(j, b):
            dst = out_hbm.at[pl.ds(base + j * CHUNK, CHUNK)]
            pltpu.make_async_copy(sbuf[b], dst, ssem[b]).wait()

        for b in range(NBUF):
            start_gather(b, b)

        def round_body(g, carry):
            for b in range(NBUF):
                j = g * NBUF + b
                wait_gather(j, b)

                @pl.when(g > 0)
                def _():
                    wait_store(j - NBUF, b)

                def row_body(i, c):
                    for r in range(4):
                        for q in range(D_MODEL // 16):
                            sl = pl.ds(q * 16, 16)
                            sbuf[b][i * 4 + r, sl] = gbuf[b][i * 4 + r, sl] * SCALE
                    return c

                lax.fori_loop(0, CHUNK // 4, row_body, 0, unroll=2)

                @pl.when(g + 1 < n_rounds)
                def _():
                    start_gather(j + NBUF, b)

                start_store(j, b)
            return carry

        lax.fori_loop(0, n_rounds, round_body, 0)

        for b in range(NBUF):
            wait_store((n_rounds - 1) * NBUF + b, b)

    return emb


def kernel(x, table):
    B = x.size
    xf = x.reshape(B).astype(jnp.int32)
    out = _emb_call(B)(xf, table)
    return out.reshape(x.shape + (D_MODEL,))
